# baseline (speedup 1.0000x reference)
import functools
import numpy as np
import jax
import jax.numpy as jnp
from jax import lax
from jax.experimental import pallas as pl
from jax.experimental.pallas import tpu as pltpu, tpu_sc as plsc

B = 4
PER = 147456
WIN = 3 * PER
TOTW = 3 * B * PER + WIN
THR = 1.15

NT = 16
SPAN = PER // (NT * 16)
CHUNK = PER // NT
PC = 1024
SP4 = PC // 16
T = NT * 16 * SP4
NR = (WIN + T - 1) // T
XCAP = 76 * 8192

VT0 = 0
ILB = 7 * PER
SMB = ILB + PER + PC
CNB = SMB + PER + PC
RCB = CNB + 2048
CRB = RCB + 256
GARB = CRB + 256
SP_SIZE = GARB + 64

_RAW = np.random.RandomState(0).randint(
    0, 2 ** 32, size=TOTW, dtype=np.uint32).astype(np.int32)

MESH = plsc.VectorSubcoreMesh(core_axis_name="c", subcore_axis_name="s")


def _lanesum(v):
    s = v[0]
    for l in range(1, 16):
        s = s + v[l]
    return s


def _prefix_vec(v, iota):
    base = jnp.zeros((16,), jnp.int32)
    for l in range(15):
        base = base + jnp.where(iota > l, v[l], 0)
    return base


@functools.partial(
    pl.kernel,
    out_type=(jax.ShapeDtypeStruct((XCAP,), jnp.float32),
              jax.ShapeDtypeStruct((XCAP,), jnp.float32)),
    mesh=MESH,
    scratch_types=(
        [pltpu.VMEM((CHUNK,), jnp.int32),
         pltpu.VMEM((CHUNK,), jnp.int32),
         pltpu.VMEM((CHUNK,), jnp.int32),
         pltpu.VMEM((128,), jnp.int32),
         pltpu.VMEM((2048,), jnp.int32),
         pltpu.VMEM((16,), jnp.int32),
         pltpu.VMEM((256,), jnp.int32)]
        + [pltpu.VMEM((PC,), jnp.int32) for _ in range(4)]
        + [pltpu.VMEM((PC,), jnp.float32) for _ in range(4)]
        + [pltpu.VMEM_SHARED((SP_SIZE,), jnp.int32),
           pltpu.SMEM((8,), jnp.int32),
           pltpu.SemaphoreType.DMA]
    ),
)
def _sc_pairs(gt_hbm, gtb_hbm, pr_hbm, seg_hbm, raw_hbm, x_hbm, w_hbm,
              segv, gidx, didx, cstg, callb, rcstg, rcall,
              iA, iB, iC, iD, fA, fB, fC, fD,
              sp, stmp, sem):
    cid = lax.axis_index("c")
    tid = lax.axis_index("s")
    iota = lax.iota(jnp.int32, 16)
    garbv = GARB + iota

    @pl.when(cid == 0)
    def _():
        @pl.loop(0, SP4)
        def _(j):
            fA[pl.ds(j * 16, 16)] = jnp.zeros((16,), jnp.float32)
        nzc = (XCAP // PC - tid + 15) >> 4

        @pl.loop(0, nzc)
        def _(k):
            c = tid + k * 16
            off = pl.multiple_of(c * PC, 8)
            pltpu.sync_copy(fA, w_hbm.at[pl.ds(off, PC)])
            pltpu.sync_copy(fA, x_hbm.at[pl.ds(off, PC)])

        @pl.loop(0, B, init_carry=(jnp.int32(0), jnp.int32(0)))
        def carry_b(b, cb):
            cursor_b, pb_b = cb
            bofs = b * PER
            tbase = bofs + tid * CHUNK

            @pl.loop(0, SPAN)
            def _(j):
                gidx[pl.ds(j * 16, 16)] = tbase + iota * SPAN + j
            pltpu.async_copy(seg_hbm.at[gidx], segv, sem).wait()
            pltpu.async_copy(gtb_hbm.at[gidx], didx, sem).wait()

            @pl.loop(0, SPAN)
            def _(j):
                s_ = pl.ds(j * 16, 16)
                segv[s_] = segv[s_] + 8 * jnp.where(didx[s_] > 0, 1, 0)

            def cbody(j, c):
                sv = segv[pl.ds(j * 16, 16)]
                di = sv >> 3
                si = sv & 7
                new = tuple(
                    c[i - 1] + jnp.where(si == i, 1, 0) * di
                    for i in range(1, 8)) + (
                    c[7] + di,)
                return new
            cnts = lax.fori_loop(
                0, SPAN, cbody,
                tuple(jnp.zeros((16,), jnp.int32) for _ in range(8)),
                unroll=4)
            for k in range(8):
                cstg[pl.ds(k * 16, 16)] = cnts[k]
            pltpu.sync_copy(cstg, sp.at[pl.ds(CNB + tid * 128, 128)])
            plsc.subcore_barrier()
            pltpu.sync_copy(sp.at[pl.ds(CNB, 2048)], callb)

            def inst_stats(k):
                tot_vec = jnp.zeros((16,), jnp.int32)
                bef_vec = jnp.zeros((16,), jnp.int32)
                for t in range(NT):
                    v = callb[pl.ds(t * 128 + k * 16, 16)]
                    tot_vec = tot_vec + v
                    bef_vec = bef_vec + jnp.where(jnp.int32(t) < tid, v, 0)
                total = _lanesum(tot_vec)
                before = _lanesum(bef_vec)
                lane_base = before + _prefix_vec(cnts[k], iota)
                return total, lane_base

            ndepth, lane_base_d = inst_stats(7)
            ic_l, nval_l, lbi_l, lbv_l = [], [], [], []
            for i in range(1, 8):
                tot, lbi = inst_stats(i - 1)
                ic_l.append(tot)
                nval_l.append(ndepth - tot)
                lbi_l.append(lbi)
                lbv_l.append(lane_base_d - lbi)
            ib_l = []
            accw = jnp.int32(ILB)
            for i in range(7):
                ib_l.append(accw)
                accw = accw + ic_l[i]

            for i in range(1, 8):
                divec = ib_l[i - 1] + lbi_l[i - 1]
                dvvec = (i - 1) * PER + lbv_l[i - 1]

                def pbody(j, c):
                    sv = segv[pl.ds(j * 16, 16)]
                    di = sv >> 3
                    mi = jnp.where((sv & 7) == i, 1, 0) * di
                    vi = di - mi
                    run_d, run_i = c
                    dest = jnp.where(
                        mi == 1, divec + run_i,
                        jnp.where(vi == 1, dvvec + (run_d - run_i), garbv))
                    didx[pl.ds(j * 16, 16)] = dest
                    return (run_d + di, run_i + mi)
                lax.fori_loop(
                    0, SPAN, pbody,
                    (jnp.zeros((16,), jnp.int32), jnp.zeros((16,), jnp.int32)),
                    unroll=4)
                pltpu.async_copy(gidx, sp.at[didx], sem).wait()
            plsc.subcore_barrier()

            @pl.loop(1, 8, init_carry=(cursor_b, pb_b))
            def carry_i(inst, ci):
                cursor, pb = ci

                def sel(vals):
                    s = jnp.int32(0)
                    for k in range(7):
                        s = s + jnp.where(inst == k + 1, vals[k], 0)
                    return s
                ic = sel(ic_l)
                nval = sel(nval_l)
                ib = sel(ib_l)
                pbi = SMB + (ib - ILB)
                vt = (inst - 1) * PER
                active = (ic > 0) & (nval > 0)
                rngm = jnp.maximum(nval - 1, 0)
                m = rngm
                for s in (1, 2, 4, 8, 16):
                    m = m | (m >> s)
                cursor_eff = jnp.minimum(cursor, TOTW - WIN)
                stmp[0] = jnp.int32(0)
                stmp[1] = jnp.int32(0)

                @pl.loop(0, NR)
                def _(r):
                    acc3 = stmp[0]
                    go = active & (rngm > 0) & (acc3 < ic)

                    @pl.when(go)
                    def _():
                        s0 = r * T
                        lane0 = (tid * 16 + iota) * SP4

                        @pl.loop(0, SP4)
                        def _(j):
                            pos = s0 + lane0 + j
                            iA[pl.ds(j * 16, 16)] = jnp.minimum(
                                cursor_eff + pos, TOTW - 1)
                        pltpu.async_copy(raw_hbm.at[iA], iB, sem).wait()

                        def r1(j, c):
                            w = iB[pl.ds(j * 16, 16)]
                            vals = w & m
                            pos = s0 + lane0 + j
                            ai = (jnp.where(vals <= rngm, 1, 0)
                                  * jnp.where(pos < WIN, 1, 0))
                            return c + ai
                        cvec = lax.fori_loop(
                            0, SP4, r1, jnp.zeros((16,), jnp.int32),
                            unroll=4)
                        rcstg[pl.ds(0, 16)] = cvec
                        pltpu.sync_copy(rcstg, sp.at[pl.ds(RCB + tid * 16, 16)])
                        plsc.subcore_barrier()
                        pltpu.sync_copy(sp.at[pl.ds(RCB, 256)], rcall)
                        rtot_vec = jnp.zeros((16,), jnp.int32)
                        bef_vec = jnp.zeros((16,), jnp.int32)
                        for t in range(NT):
                            v = rcall[pl.ds(t * 16, 16)]
                            rtot_vec = rtot_vec + v
                            bef_vec = bef_vec + jnp.where(jnp.int32(t) < tid, v, 0)
                        rtot = _lanesum(rtot_vec)
                        lane_base = acc3 + _lanesum(bef_vec) + _prefix_vec(cvec, iota)

                        def r2(j, c):
                            run, consv = c
                            w = iB[pl.ds(j * 16, 16)]
                            vals = w & m
                            pos = s0 + lane0 + j
                            ai = (jnp.where(vals <= rngm, 1, 0)
                                  * jnp.where(pos < WIN, 1, 0))
                            rank = lane_base + run
                            q1 = ai * jnp.where(rank < ic, 1, 0)
                            q2 = ai * jnp.where(rank == ic - 1, 1, 0)
                            iC[pl.ds(j * 16, 16)] = jnp.where(
                                q1 == 1, pbi + rank, garbv)
                            iD[pl.ds(j * 16, 16)] = vals
                            return (run + ai, consv + q2 * pos)
                        _, consv = lax.fori_loop(
                            0, SP4, r2,
                            (jnp.zeros((16,), jnp.int32),
                             jnp.zeros((16,), jnp.int32)),
                            unroll=4)
                        pltpu.async_copy(iD, sp.at[iC], sem).wait()
                        stmp[0] = acc3 + rtot
                        stmp[1] = stmp[1] + _lanesum(consv)
                        plsc.subcore_barrier()

                rcstg[pl.ds(0, 16)] = jnp.full((16,), 0, jnp.int32) + stmp[1]
                pltpu.sync_copy(rcstg, sp.at[pl.ds(CRB + tid * 16, 16)])
                plsc.subcore_barrier()
                pltpu.sync_copy(sp.at[pl.ds(CRB, 256)], rcall)
                cons_sum = jnp.int32(0)
                for t in range(NT):
                    cons_sum = cons_sum + rcall[pl.ds(t * 16, 16)][0]
                plsc.subcore_barrier()
                acc_fin = stmp[0]
                consumed = jnp.where(acc_fin >= ic, cons_sum + 1, WIN + 1)
                cursor_new = cursor + jnp.where(active & (rngm > 0), consumed, 0)

                nch = jnp.where(active, (ic + PC - 1) >> 10, 0)
                nck = (nch - tid + 15) >> 4

                @pl.loop(0, nck)
                def _(kk):
                    c = tid + kk * 16
                    r0 = c * PC
                    lane0 = iota * SP4

                    @pl.loop(0, SP4)
                    def _(j):
                        iA[pl.ds(j * 16, 16)] = ib + r0 + lane0 + j
                    pltpu.async_copy(sp.at[iA], iB, sem).wait()

                    @pl.loop(0, SP4)
                    def _(j):
                        iA[pl.ds(j * 16, 16)] = pbi + r0 + lane0 + j
                    pltpu.async_copy(sp.at[iA], iC, sem).wait()

                    @pl.loop(0, SP4)
                    def _(j):
                        s_ = pl.ds(j * 16, 16)
                        sv = iC[s_]
                        se = jnp.where(rngm > 0, sv, 0)
                        sc = jnp.minimum(jnp.maximum(se, 0), nval - 1)
                        iA[s_] = vt + sc
                        ip = iB[s_]
                        iB[s_] = jnp.minimum(jnp.maximum(ip, 0), B * PER - 1)
                    pltpu.async_copy(sp.at[iA], iD, sem).wait()

                    @pl.loop(0, SP4)
                    def _(j):
                        s_ = pl.ds(j * 16, 16)
                        sv = iC[s_]
                        se = jnp.where(rngm > 0, sv, 0)
                        jp = jnp.where(se < nval, iD[s_], bofs + PER - 1)
                        iA[s_] = jnp.minimum(jnp.maximum(jp, 0), B * PER - 1)
                    c1 = pltpu.async_copy(gt_hbm.at[iB], fA, sem)
                    c2 = pltpu.async_copy(gt_hbm.at[iA], fB, sem)
                    c3 = pltpu.async_copy(pr_hbm.at[iB], fC, sem)
                    c4 = pltpu.async_copy(pr_hbm.at[iA], fD, sem)
                    c1.wait()
                    c2.wait()
                    c3.wait()
                    c4.wait()

                    @pl.loop(0, SP4)
                    def _(j):
                        s_ = pl.ds(j * 16, 16)
                        a = fA[s_]
                        b_ = fB[s_]
                        p1 = fC[s_]
                        p2 = fD[s_]
                        t1 = jnp.where(a / b_ >= THR, 1.0, 0.0).astype(jnp.float32)
                        tg = jnp.where(b_ / a > THR, -1.0, t1).astype(jnp.float32)
                        rank = r0 + lane0 + j
                        oki = (jnp.where(tg != 0.0, 1, 0)
                               * jnp.where(rank < ic, 1, 0))
                        fA[s_] = jnp.where(
                            oki == 1, -tg * (p1 - p2), 0.0).astype(jnp.float32)
                        fB[s_] = oki.astype(jnp.float32)
                    off = pl.multiple_of(pb + r0, 8)
                    pltpu.sync_copy(fA, x_hbm.at[pl.ds(off, PC)])
                    pltpu.sync_copy(fB, w_hbm.at[pl.ds(off, PC)])

                pb_new = pb + jnp.where(active, nch * PC, 0)
                return (cursor_new, pb_new)
            return carry_i


_NBLK = XCAP // 8192


def _red_body(x_ref, w_ref, o_ref, acc):
    @pl.when(pl.program_id(0) == 0)
    def _():
        acc[0] = jnp.float32(0.0)
        acc[1] = jnp.float32(0.0)
    xv = x_ref[0]
    wv = w_ref[0]
    term = jnp.log(1.0 + jnp.exp(xv)) * wv
    acc[0] = acc[0] + jnp.sum(term)
    acc[1] = acc[1] + jnp.sum(wv)

    @pl.when(pl.program_id(0) == _NBLK - 1)
    def _():
        tot = acc[0]
        cnt = acc[1]
        loss = tot / jnp.maximum(cnt, jnp.float32(1.0))
        o_ref[...] = jnp.full((8, 128), loss, jnp.float32)


def _tc_reduce(x, w):
    out = pl.pallas_call(
        _red_body,
        grid=(_NBLK,),
        in_specs=[pl.BlockSpec((1, 8, 1024), lambda i: (i, 0, 0)),
                  pl.BlockSpec((1, 8, 1024), lambda i: (i, 0, 0))],
        out_specs=pl.BlockSpec((8, 128), lambda i: (0, 0)),
        out_shape=jax.ShapeDtypeStruct((8, 128), jnp.float32),
        scratch_shapes=[pltpu.SMEM((2,), jnp.float32)],
    )(x.reshape(_NBLK, 8, 1024), w.reshape(_NBLK, 8, 1024))
    return out[0, 0]


@jax.jit
def _run(pred_depth, gt_depth, seg_masks):
    gt_flat = gt_depth.reshape(B * PER)
    pr_flat = pred_depth.reshape(B * PER)
    seg_flat = seg_masks.reshape(B * PER)
    gtb_flat = lax.bitcast_convert_type(gt_flat, jnp.int32)
    raw = jnp.asarray(_RAW)
    x, w = _sc_pairs(gt_flat, gtb_flat, pr_flat, seg_flat, raw)
    return _tc_reduce(x, w).reshape(1).astype(jnp.float32)


def kernel(pred_depth, gt_depth, seg_masks):
    return _run(pred_depth, gt_depth, seg_masks)


# speedup vs baseline: 1.0574x; 1.0574x over previous
import functools
import numpy as np
import jax
import jax.numpy as jnp
from jax import lax
from jax.experimental import pallas as pl
from jax.experimental.pallas import tpu as pltpu, tpu_sc as plsc

B = 4
PER = 147456
WIN = 3 * PER
TOTW = 3 * B * PER + WIN
THR = 1.15

NT = 16
SPAN = PER // (NT * 16)
CHUNK = PER // NT
PC = 1024
SP4 = PC // 16
T = NT * 16 * SP4
NR = (WIN + T - 1) // T
XCAP = 76 * 8192

VT0 = 0
ILB = 7 * PER
SMB = ILB + PER + PC
CNB = SMB + PER + PC
RCB = CNB + 2048
CRB = RCB + 256
GARB = CRB + 256
SP_SIZE = GARB + 64

_RAW = np.random.RandomState(0).randint(
    0, 2 ** 32, size=TOTW, dtype=np.uint32).astype(np.int32)

MESH = plsc.VectorSubcoreMesh(core_axis_name="c", subcore_axis_name="s")


def _lanesum(v):
    s = v[0]
    for l in range(1, 16):
        s = s + v[l]
    return s


def _prefix_vec(v, iota):
    base = jnp.zeros((16,), jnp.int32)
    for l in range(15):
        base = base + jnp.where(iota > l, v[l], 0)
    return base


@functools.partial(
    pl.kernel,
    out_type=(jax.ShapeDtypeStruct((XCAP,), jnp.float32),
              jax.ShapeDtypeStruct((XCAP,), jnp.float32)),
    mesh=MESH,
    scratch_types=(
        [pltpu.VMEM((CHUNK,), jnp.int32),
         pltpu.VMEM((CHUNK,), jnp.int32),
         pltpu.VMEM((CHUNK,), jnp.int32),
         pltpu.VMEM((128,), jnp.int32),
         pltpu.VMEM((2048,), jnp.int32),
         pltpu.VMEM((16,), jnp.int32),
         pltpu.VMEM((256,), jnp.int32)]
        + [pltpu.VMEM((PC,), jnp.int32) for _ in range(4)]
        + [pltpu.VMEM((PC,), jnp.float32) for _ in range(4)]
        + [pltpu.VMEM_SHARED((SP_SIZE,), jnp.int32),
           pltpu.SMEM((8,), jnp.int32),
           pltpu.SemaphoreType.DMA]
    ),
)
def _sc_pairs(gt_hbm, gtb_hbm, pr_hbm, seg_hbm, raw_hbm, x_hbm, w_hbm,
              segv, gidx, didx, cstg, callb, rcstg, rcall,
              iA, iB, iC, iD, fA, fB, fC, fD,
              sp, stmp, sem):
    cid = lax.axis_index("c")
    tid = lax.axis_index("s")
    iota = lax.iota(jnp.int32, 16)
    garbv = GARB + iota

    @pl.when(cid == 0)
    def _():
        @pl.loop(0, SP4)
        def _(j):
            fA[pl.ds(j * 16, 16)] = jnp.zeros((16,), jnp.float32)
        nzc = (XCAP // PC - tid + 15) >> 4

        @pl.loop(0, nzc)
        def _(k):
            c = tid + k * 16
            off = pl.multiple_of(c * PC, 8)
            pltpu.sync_copy(fA, w_hbm.at[pl.ds(off, PC)])
            pltpu.sync_copy(fA, x_hbm.at[pl.ds(off, PC)])

        @pl.loop(0, B, init_carry=(jnp.int32(0), jnp.int32(0)))
        def carry_b(b, cb):
            cursor_b, pb_b = cb
            bofs = b * PER
            tbase = bofs + tid * CHUNK

            @pl.loop(0, SPAN)
            def _(j):
                gidx[pl.ds(j * 16, 16)] = tbase + iota * SPAN + j
            pltpu.async_copy(seg_hbm.at[gidx], segv, sem).wait()
            pltpu.async_copy(gtb_hbm.at[gidx], didx, sem).wait()

            @pl.loop(0, SPAN)
            def _(j):
                s_ = pl.ds(j * 16, 16)
                segv[s_] = segv[s_] + 8 * jnp.where(didx[s_] > 0, 1, 0)

            def cbody(j, c):
                sv = segv[pl.ds(j * 16, 16)]
                di = sv >> 3
                si = sv & 7
                new = tuple(
                    c[i - 1] + jnp.where(si == i, 1, 0) * di
                    for i in range(1, 8)) + (
                    c[7] + di,)
                return new
            cnts = lax.fori_loop(
                0, SPAN, cbody,
                tuple(jnp.zeros((16,), jnp.int32) for _ in range(8)))
            for k in range(8):
                cstg[pl.ds(k * 16, 16)] = cnts[k]
            pltpu.sync_copy(cstg, sp.at[pl.ds(CNB + tid * 128, 128)])
            plsc.subcore_barrier()
            pltpu.sync_copy(sp.at[pl.ds(CNB, 2048)], callb)

            def inst_stats(k):
                tot_vec = jnp.zeros((16,), jnp.int32)
                bef_vec = jnp.zeros((16,), jnp.int32)
                for t in range(NT):
                    v = callb[pl.ds(t * 128 + k * 16, 16)]
                    tot_vec = tot_vec + v
                    bef_vec = bef_vec + jnp.where(jnp.int32(t) < tid, v, 0)
                total = _lanesum(tot_vec)
                before = _lanesum(bef_vec)
                lane_base = before + _prefix_vec(cnts[k], iota)
                return total, lane_base

            ndepth, lane_base_d = inst_stats(7)
            ic_l, nval_l, lbi_l, lbv_l = [], [], [], []
            for i in range(1, 8):
                tot, lbi = inst_stats(i - 1)
                ic_l.append(tot)
                nval_l.append(ndepth - tot)
                lbi_l.append(lbi)
                lbv_l.append(lane_base_d - lbi)
            ib_l = []
            accw = jnp.int32(ILB)
            for i in range(7):
                ib_l.append(accw)
                accw = accw + ic_l[i]

            for i in range(1, 8):
                divec = ib_l[i - 1] + lbi_l[i - 1]
                dvvec = (i - 1) * PER + lbv_l[i - 1]

                def pbody(j, c):
                    sv = segv[pl.ds(j * 16, 16)]
                    di = sv >> 3
                    mi = jnp.where((sv & 7) == i, 1, 0) * di
                    vi = di - mi
                    run_d, run_i = c
                    dest = jnp.where(
                        mi == 1, divec + run_i,
                        jnp.where(vi == 1, dvvec + (run_d - run_i), garbv))
                    didx[pl.ds(j * 16, 16)] = dest
                    return (run_d + di, run_i + mi)
                lax.fori_loop(
                    0, SPAN, pbody,
                    (jnp.zeros((16,), jnp.int32), jnp.zeros((16,), jnp.int32)))
                pltpu.async_copy(gidx, sp.at[didx], sem).wait()
            plsc.subcore_barrier()

            @pl.loop(1, 8, init_carry=(cursor_b, pb_b))
            def carry_i(inst, ci):
                cursor, pb = ci

                def sel(vals):
                    s = jnp.int32(0)
                    for k in range(7):
                        s = s + jnp.where(inst == k + 1, vals[k], 0)
                    return s
                ic = sel(ic_l)
                nval = sel(nval_l)
                ib = sel(ib_l)
                pbi = SMB + (ib - ILB)
                vt = (inst - 1) * PER
                active = (ic > 0) & (nval > 0)
                rngm = jnp.maximum(nval - 1, 0)
                m = rngm
                for s in (1, 2, 4, 8, 16):
                    m = m | (m >> s)
                cursor_eff = jnp.minimum(cursor, TOTW - WIN)
                stmp[0] = jnp.int32(0)
                stmp[1] = jnp.int32(0)

                @pl.loop(0, NR)
                def _(r):
                    acc3 = stmp[0]
                    go = active & (rngm > 0) & (acc3 < ic)

                    @pl.when(go)
                    def _():
                        s0 = r * T
                        lane0 = (tid * 16 + iota) * SP4

                        @pl.loop(0, SP4)
                        def _(j):
                            pos = s0 + lane0 + j
                            iA[pl.ds(j * 16, 16)] = jnp.minimum(
                                cursor_eff + pos, TOTW - 1)
                        pltpu.async_copy(raw_hbm.at[iA], iB, sem).wait()

                        def r1(j, c):
                            w = iB[pl.ds(j * 16, 16)]
                            vals = w & m
                            pos = s0 + lane0 + j
                            ai = (jnp.where(vals <= rngm, 1, 0)
                                  * jnp.where(pos < WIN, 1, 0))
                            return c + ai
                        cvec = lax.fori_loop(
                            0, SP4, r1, jnp.zeros((16,), jnp.int32))
                        rcstg[pl.ds(0, 16)] = cvec
                        pltpu.sync_copy(rcstg, sp.at[pl.ds(RCB + tid * 16, 16)])
                        plsc.subcore_barrier()
                        pltpu.sync_copy(sp.at[pl.ds(RCB, 256)], rcall)
                        rtot_vec = jnp.zeros((16,), jnp.int32)
                        bef_vec = jnp.zeros((16,), jnp.int32)
                        for t in range(NT):
                            v = rcall[pl.ds(t * 16, 16)]
                            rtot_vec = rtot_vec + v
                            bef_vec = bef_vec + jnp.where(jnp.int32(t) < tid, v, 0)
                        rtot = _lanesum(rtot_vec)
                        lane_base = acc3 + _lanesum(bef_vec) + _prefix_vec(cvec, iota)

                        def r2(j, c):
                            run, consv = c
                            w = iB[pl.ds(j * 16, 16)]
                            vals = w & m
                            pos = s0 + lane0 + j
                            ai = (jnp.where(vals <= rngm, 1, 0)
                                  * jnp.where(pos < WIN, 1, 0))
                            rank = lane_base + run
                            q1 = ai * jnp.where(rank < ic, 1, 0)
                            q2 = ai * jnp.where(rank == ic - 1, 1, 0)
                            iC[pl.ds(j * 16, 16)] = jnp.where(
                                q1 == 1, pbi + rank, garbv)
                            iD[pl.ds(j * 16, 16)] = vals
                            return (run + ai, consv + q2 * pos)
                        _, consv = lax.fori_loop(
                            0, SP4, r2,
                            (jnp.zeros((16,), jnp.int32),
                             jnp.zeros((16,), jnp.int32)))
                        pltpu.async_copy(iD, sp.at[iC], sem).wait()
                        stmp[0] = acc3 + rtot
                        stmp[1] = stmp[1] + _lanesum(consv)
                        plsc.subcore_barrier()

                rcstg[pl.ds(0, 16)] = jnp.full((16,), 0, jnp.int32) + stmp[1]
                pltpu.sync_copy(rcstg, sp.at[pl.ds(CRB + tid * 16, 16)])
                plsc.subcore_barrier()
                pltpu.sync_copy(sp.at[pl.ds(CRB, 256)], rcall)
                cons_sum = jnp.int32(0)
                for t in range(NT):
                    cons_sum = cons_sum + rcall[pl.ds(t * 16, 16)][0]
                plsc.subcore_barrier()
                acc_fin = stmp[0]
                consumed = jnp.where(acc_fin >= ic, cons_sum + 1, WIN + 1)
                cursor_new = cursor + jnp.where(active & (rngm > 0), consumed, 0)

                nch = jnp.where(active, (ic + PC - 1) >> 10, 0)
                nck = (nch - tid + 15) >> 4

                @pl.loop(0, nck)
                def _(kk):
                    c = tid + kk * 16
                    r0 = c * PC
                    lane0 = iota * SP4

                    @pl.loop(0, SP4)
                    def _(j):
                        iA[pl.ds(j * 16, 16)] = ib + r0 + lane0 + j
                    pltpu.async_copy(sp.at[iA], iB, sem).wait()

                    @pl.loop(0, SP4)
                    def _(j):
                        iA[pl.ds(j * 16, 16)] = pbi + r0 + lane0 + j
                    pltpu.async_copy(sp.at[iA], iC, sem).wait()

                    @pl.loop(0, SP4)
                    def _(j):
                        s_ = pl.ds(j * 16, 16)
                        sv = iC[s_]
                        se = jnp.where(rngm > 0, sv, 0)
                        sc = jnp.minimum(jnp.maximum(se, 0), nval - 1)
                        iA[s_] = vt + sc
                        ip = iB[s_]
                        iB[s_] = jnp.minimum(jnp.maximum(ip, 0), B * PER - 1)
                    pltpu.async_copy(sp.at[iA], iD, sem).wait()

                    @pl.loop(0, SP4)
                    def _(j):
                        s_ = pl.ds(j * 16, 16)
                        sv = iC[s_]
                        se = jnp.where(rngm > 0, sv, 0)
                        jp = jnp.where(se < nval, iD[s_], bofs + PER - 1)
                        iA[s_] = jnp.minimum(jnp.maximum(jp, 0), B * PER - 1)
                    c1 = pltpu.async_copy(gt_hbm.at[iB], fA, sem)
                    c2 = pltpu.async_copy(gt_hbm.at[iA], fB, sem)
                    c3 = pltpu.async_copy(pr_hbm.at[iB], fC, sem)
                    c4 = pltpu.async_copy(pr_hbm.at[iA], fD, sem)
                    c1.wait()
                    c2.wait()
                    c3.wait()
                    c4.wait()

                    @pl.loop(0, SP4)
                    def _(j):
                        s_ = pl.ds(j * 16, 16)
                        a = fA[s_]
                        b_ = fB[s_]
                        p1 = fC[s_]
                        p2 = fD[s_]
                        t1 = jnp.where(a / b_ >= THR, 1.0, 0.0).astype(jnp.float32)
                        tg = jnp.where(b_ / a > THR, -1.0, t1).astype(jnp.float32)
                        rank = r0 + lane0 + j
                        oki = (jnp.where(tg != 0.0, 1, 0)
                               * jnp.where(rank < ic, 1, 0))
                        fA[s_] = jnp.where(
                            oki == 1, -tg * (p1 - p2), 0.0).astype(jnp.float32)
                        fB[s_] = oki.astype(jnp.float32)
                    off = pl.multiple_of(pb + r0, 8)
                    pltpu.sync_copy(fA, x_hbm.at[pl.ds(off, PC)])
                    pltpu.sync_copy(fB, w_hbm.at[pl.ds(off, PC)])

                pb_new = pb + jnp.where(active, nch * PC, 0)
                return (cursor_new, pb_new)
            return carry_i


_NBLK = XCAP // 8192


def _red_body(x_ref, w_ref, o_ref, acc):
    @pl.when(pl.program_id(0) == 0)
    def _():
        acc[0] = jnp.float32(0.0)
        acc[1] = jnp.float32(0.0)
    xv = x_ref[0]
    wv = w_ref[0]
    term = jnp.log(1.0 + jnp.exp(xv)) * wv
    acc[0] = acc[0] + jnp.sum(term)
    acc[1] = acc[1] + jnp.sum(wv)

    @pl.when(pl.program_id(0) == _NBLK - 1)
    def _():
        tot = acc[0]
        cnt = acc[1]
        loss = tot / jnp.maximum(cnt, jnp.float32(1.0))
        o_ref[...] = jnp.full((8, 128), loss, jnp.float32)


def _tc_reduce(x, w):
    out = pl.pallas_call(
        _red_body,
        grid=(_NBLK,),
        in_specs=[pl.BlockSpec((1, 8, 1024), lambda i: (i, 0, 0)),
                  pl.BlockSpec((1, 8, 1024), lambda i: (i, 0, 0))],
        out_specs=pl.BlockSpec((8, 128), lambda i: (0, 0)),
        out_shape=jax.ShapeDtypeStruct((8, 128), jnp.float32),
        scratch_shapes=[pltpu.SMEM((2,), jnp.float32)],
    )(x.reshape(_NBLK, 8, 1024), w.reshape(_NBLK, 8, 1024))
    return out[0, 0]


@jax.jit
def _run(pred_depth, gt_depth, seg_masks):
    gt_flat = gt_depth.reshape(B * PER)
    pr_flat = pred_depth.reshape(B * PER)
    seg_flat = seg_masks.reshape(B * PER)
    gtb_flat = lax.bitcast_convert_type(gt_flat, jnp.int32)
    raw = jnp.asarray(_RAW)
    x, w = _sc_pairs(gt_flat, gtb_flat, pr_flat, seg_flat, raw)
    return _tc_reduce(x, w).reshape(1).astype(jnp.float32)


def kernel(pred_depth, gt_depth, seg_masks):
    return _run(pred_depth, gt_depth, seg_masks)
